# baseline (speedup 1.0000x reference)
import functools
import math

import jax
import jax.numpy as jnp
from jax import lax
from jax.experimental import pallas as pl
from jax.experimental.pallas import tpu as pltpu
from jax.experimental.pallas import tpu_sc as plsc

_EMB = 64
_SCALE = math.sqrt(_EMB)
_LANES = 16


@functools.lru_cache(maxsize=None)
def _build(B0, T, V, D):
    NC, NS = 2, 16
    NW = NC * NS
    B = B0 * T
    b_per_w = B // NW
    C = 256
    CB = C // 128
    assert B0 % C == 0 and b_per_w % C == 0
    n_chunks = b_per_w // C
    chunks_per_t = B0 // C
    half = n_chunks // 2
    assert n_chunks % 2 == 0
    DT = D // 8

    mesh = plsc.VectorSubcoreMesh(core_axis_name="c", subcore_axis_name="s")

    @functools.partial(
        pl.kernel,
        mesh=mesh,
        out_type=jax.ShapeDtypeStruct((T, DT, B0 // 128, 8 * 128), jnp.float32),
        scratch_types=[
            pltpu.VMEM((b_per_w,), jnp.int32),
            pltpu.VMEM((C, D), jnp.float32),
            pltpu.VMEM((C, D), jnp.float32),
            pltpu.VMEM((DT, CB, 8 * 128), jnp.float32),
            pltpu.VMEM((DT, CB, 8 * 128), jnp.float32),
            pltpu.VMEM((16, 16), jnp.int32),
            pltpu.VMEM((16, 16), jnp.int32),
            pltpu.VMEM((16, 16), jnp.int32),
            pltpu.SemaphoreType.DMA,
            pltpu.SemaphoreType.DMA,
            pltpu.SemaphoreType.DMA,
            pltpu.SemaphoreType.DMA,
        ],
        compiler_params=pltpu.CompilerParams(
            use_tc_tiling_on_sc=False, needs_layout_passes=False),
    )
    def emb_kernel(table_hbm, tok_hbm, out_hbm, idx_v, g0, g1, s0, s1,
                   ptab, itab, etab, sg0, sg1, ss0, ss1):
        wid = lax.axis_index("s") * NC + lax.axis_index("c")
        base = wid * n_chunks
        gb, sb = (g0, g1), (s0, s1)
        sems_g, sems_s = (sg0, sg1), (ss0, ss1)
        iota16 = lax.iota(jnp.int32, 16)

        def tab_body(sh, carry):
            p = (iota16 + sh) & 15
            ptab[sh, :] = p
            itab[sh, :] = p >> 3
            etab[sh, :] = (p & 7) * 128 + iota16
            return carry

        lax.fori_loop(0, 16, tab_body, 0)

        pltpu.sync_copy(tok_hbm.at[pl.ds(base * C, b_per_w)], idx_v)

        def gather(k, b):
            return pltpu.make_async_copy(
                table_hbm.at[idx_v.at[pl.ds(k * C, C)]], gb[b], sems_g[b])

        def scatter(k, b):
            cid = base + k
            t = cid // chunks_per_t
            bb = (cid % chunks_per_t) * CB
            return pltpu.make_async_copy(
                sb[b], out_hbm.at[t, :, pl.ds(bb, CB), :], sems_s[b])

        jvecs = [jnp.full((16,), j, jnp.int32) for j in range(CB)]

        def transpose_scale(b):
            g, s = gb[b], sb[b]

            @plsc.parallel_loop(0, 16, step=1, unroll=2)
            def _(sh):
                p = ptab[sh, :]
                iad = itab[sh, :]
                eb = etab[sh, :]
                for kk in range(128 // _LANES):
                    for j in range(CB):
                        rows = iota16 + (j * 128 + kk * _LANES)
                        eb_k = eb + kk * _LANES
                        for d0 in range(0, D, 16):
                            cols = p + d0
                            v = plsc.load_gather(g, [rows, cols])
                            plsc.store_scatter(
                                s, [iad + (d0 >> 3), jvecs[j], eb_k],
                                v * _SCALE)

        gather(0, 0).start()
        gather(1, 1).start()

        def ring(tt, carry):
            for b in range(2):
                k = 2 * tt + b
                gather(k, b).wait()

                @pl.when(tt > 0)
                def _():
                    scatter(k - 2, b).wait()

                transpose_scale(b)
                scatter(k, b).start()

                @pl.when(tt < half - 1)
                def _():
                    gather(k + 2, b).start()

            return carry

        lax.fori_loop(0, half, ring, 0)

        for b in range(2):
            scatter(n_chunks - 2 + b, b).wait()

    return emb_kernel


def kernel(tokens, table):
    B0, T = tokens.shape
    V, D = table.shape
    flat = tokens.T.reshape(B0 * T).astype(jnp.int32)
    out4 = _build(B0, T, V, D)(table, flat)
    out = (
        out4.reshape(T, D // 8, B0 // 128, 8, 128)
        .transpose(2, 4, 0, 1, 3)
        .reshape(B0, T, D)
    )
    return out


# speedup vs baseline: 2.5732x; 2.5732x over previous
import functools
import math

import jax
import jax.numpy as jnp
from jax import lax
from jax.experimental import pallas as pl
from jax.experimental.pallas import tpu as pltpu
from jax.experimental.pallas import tpu_sc as plsc

_EMB = 64
_SCALE = math.sqrt(_EMB)
_LANES = 16


@functools.lru_cache(maxsize=None)
def _build(B0, T, V, D):
    NC, NS = 2, 16
    NW = NC * NS
    B = B0 * T
    b_per_w = B // NW
    C = 256
    CB = C // 128
    assert B0 % C == 0 and b_per_w % C == 0
    n_chunks = b_per_w // C
    chunks_per_t = B0 // C
    half = n_chunks // 2
    assert n_chunks % 2 == 0
    DT = D // 8

    mesh = plsc.VectorSubcoreMesh(core_axis_name="c", subcore_axis_name="s")

    @functools.partial(
        pl.kernel,
        mesh=mesh,
        out_type=jax.ShapeDtypeStruct((T, DT, B0 // 128, 8 * 128), jnp.float32),
        scratch_types=[
            pltpu.VMEM((b_per_w,), jnp.int32),
            pltpu.VMEM((C, D), jnp.float32),
            pltpu.VMEM((C, D), jnp.float32),
            pltpu.VMEM((DT, CB, 8 * 128), jnp.float32),
            pltpu.VMEM((DT, CB, 8 * 128), jnp.float32),
            pltpu.VMEM((16, 16), jnp.int32),
            pltpu.VMEM((16, 16), jnp.int32),
            pltpu.VMEM((16, 16), jnp.int32),
            pltpu.SemaphoreType.DMA,
            pltpu.SemaphoreType.DMA,
            pltpu.SemaphoreType.DMA,
            pltpu.SemaphoreType.DMA,
        ],
        compiler_params=pltpu.CompilerParams(
            use_tc_tiling_on_sc=False, needs_layout_passes=False),
    )
    def emb_kernel(table_hbm, tok_hbm, out_hbm, idx_v, g0, g1, s0, s1,
                   ptab, itab, etab, sg0, sg1, ss0, ss1):
        wid = lax.axis_index("s") * NC + lax.axis_index("c")
        base = wid * n_chunks
        gb, sb = (g0, g1), (s0, s1)
        sems_g, sems_s = (sg0, sg1), (ss0, ss1)
        iota16 = lax.iota(jnp.int32, 16)

        def tab_body(sh, carry):
            p = (iota16 + sh) & 15
            ptab[sh, :] = p
            itab[sh, :] = p >> 3
            etab[sh, :] = (p & 7) * 128 + iota16
            return carry

        lax.fori_loop(0, 16, tab_body, 0)

        pltpu.sync_copy(tok_hbm.at[pl.ds(base * C, b_per_w)], idx_v)

        def gather(k, b):
            return pltpu.make_async_copy(
                table_hbm.at[idx_v.at[pl.ds(k * C, C)]], gb[b], sems_g[b])

        def scatter(k, b):
            cid = base + k
            t = cid // chunks_per_t
            bb = (cid % chunks_per_t) * CB
            return pltpu.make_async_copy(
                sb[b], out_hbm.at[t, :, pl.ds(bb, CB), :], sems_s[b])

        jvecs = [jnp.full((16,), j, jnp.int32) for j in range(CB)]

        def transpose_scale(b):
            g, s = gb[b], sb[b]

            def sh_body(sh, carry):
                p = ptab[sh, :]
                iad = itab[sh, :]
                eb = etab[sh, :]
                iads = [iad + i for i in range(DT)]
                for kk in range(128 // _LANES):
                    eb_k = eb + kk * _LANES
                    vals, sidx = [], []
                    for j in range(CB):
                        rows = iota16 + (j * 128 + kk * _LANES)
                        for d0 in range(0, D, 16):
                            v = plsc.load_gather(g, [rows, p + d0])
                            vals.append(v * _SCALE)
                            sidx.append((iads[d0 >> 3], jvecs[j], eb_k))
                    for v, (ii, jj, ee) in zip(vals, sidx):
                        plsc.store_scatter(s, [ii, jj, ee], v)
                return carry

            lax.fori_loop(0, 16, sh_body, 0)

        gather(0, 0).start()
        gather(1, 1).start()

        def ring(tt, carry):
            for b in range(2):
                k = 2 * tt + b
                gather(k, b).wait()

                @pl.when(tt > 0)
                def _():
                    scatter(k - 2, b).wait()

                transpose_scale(b)
                scatter(k, b).start()

                @pl.when(tt < half - 1)
                def _():
                    gather(k + 2, b).start()

            return carry

        lax.fori_loop(0, half, ring, 0)

        for b in range(2):
            scatter(n_chunks - 2 + b, b).wait()

    return emb_kernel


def kernel(tokens, table):
    B0, T = tokens.shape
    V, D = table.shape
    flat = tokens.T.reshape(B0 * T).astype(jnp.int32)
    out4 = _build(B0, T, V, D)(table, flat)
    out = (
        out4.reshape(T, D // 8, B0 // 128, 8, 128)
        .transpose(2, 4, 0, 1, 3)
        .reshape(B0, T, D)
    )
    return out


# speedup vs baseline: 3.1091x; 1.2082x over previous
import functools
import math

import jax
import jax.numpy as jnp
from jax import lax
from jax.experimental import pallas as pl
from jax.experimental.pallas import tpu as pltpu
from jax.experimental.pallas import tpu_sc as plsc

_EMB = 64
_SCALE = math.sqrt(_EMB)
_LANES = 16


@functools.lru_cache(maxsize=None)
def _build(B0, T, V, D):
    NC, NS = 2, 16
    NW = NC * NS
    B = B0 * T
    b_per_w = B // NW
    C = 256
    CB = C // 128
    assert B0 % C == 0 and b_per_w % C == 0
    n_chunks = b_per_w // C
    chunks_per_t = B0 // C
    half = n_chunks // 2
    assert n_chunks % 2 == 0
    DT = D // 8

    mesh = plsc.VectorSubcoreMesh(core_axis_name="c", subcore_axis_name="s")

    @functools.partial(
        pl.kernel,
        mesh=mesh,
        out_type=jax.ShapeDtypeStruct((T, DT, B0 // 128, 8 * 128), jnp.float32),
        scratch_types=[
            pltpu.VMEM((b_per_w,), jnp.int32),
            pltpu.VMEM((C, D), jnp.float32),
            pltpu.VMEM((C, D), jnp.float32),
            pltpu.VMEM((DT, CB, 8 * 128), jnp.float32),
            pltpu.VMEM((DT, CB, 8 * 128), jnp.float32),
            pltpu.VMEM((16, 16), jnp.int32),
            pltpu.VMEM((16, 16), jnp.int32),
            pltpu.VMEM((16, 16), jnp.int32),
            pltpu.SemaphoreType.DMA,
            pltpu.SemaphoreType.DMA,
            pltpu.SemaphoreType.DMA,
            pltpu.SemaphoreType.DMA,
        ],
        compiler_params=pltpu.CompilerParams(
            use_tc_tiling_on_sc=False, needs_layout_passes=False),
    )
    def emb_kernel(table_hbm, tok_hbm, out_hbm, idx_v, g0, g1, s0, s1,
                   ptab, itab, etab, sg0, sg1, ss0, ss1):
        wid = lax.axis_index("s") * NC + lax.axis_index("c")
        base = wid * n_chunks
        gb, sb = (g0, g1), (s0, s1)
        sems_g, sems_s = (sg0, sg1), (ss0, ss1)
        iota16 = lax.iota(jnp.int32, 16)

        def tab_body(sh, carry):
            p = (iota16 + sh) & 15
            ptab[sh, :] = p
            itab[sh, :] = p >> 3
            etab[sh, :] = (p & 7) * 128 + iota16
            return carry

        lax.fori_loop(0, 16, tab_body, 0)

        pltpu.sync_copy(tok_hbm.at[pl.ds(base * C, b_per_w)], idx_v)

        def gather(k, b):
            return pltpu.make_async_copy(
                table_hbm.at[idx_v.at[pl.ds(k * C, C)]], gb[b], sems_g[b])

        def scatter(k, b):
            cid = base + k
            t = cid // chunks_per_t
            bb = (cid % chunks_per_t) * CB
            return pltpu.make_async_copy(
                sb[b], out_hbm.at[t, :, pl.ds(bb, CB), :], sems_s[b])

        jvecs = [jnp.full((16,), j, jnp.int32) for j in range(CB)]

        def transpose_scale(b):
            g, s = gb[b], sb[b]

            def sh_body(sh, carry):
                p = ptab[sh, :]
                iad = itab[sh, :]
                eb = etab[sh, :]
                iads = [iad + i for i in range(DT)]
                for kk2 in range(128 // _LANES // 2):
                    vals, sidx = [], []
                    for kk in (2 * kk2, 2 * kk2 + 1):
                        eb_k = eb + kk * _LANES
                        for j in range(CB):
                            rows = iota16 + (j * 128 + kk * _LANES)
                            for d0 in range(0, D, 16):
                                v = plsc.load_gather(g, [rows, p + d0])
                                vals.append(v * _SCALE)
                                sidx.append((iads[d0 >> 3], jvecs[j], eb_k))
                    for v, (ii, jj, ee) in zip(vals, sidx):
                        plsc.store_scatter(s, [ii, jj, ee], v)
                return carry

            lax.fori_loop(0, 16, sh_body, 0)

        gather(0, 0).start()
        gather(1, 1).start()

        def ring(tt, carry):
            for b in range(2):
                k = 2 * tt + b
                gather(k, b).wait()

                @pl.when(tt > 0)
                def _():
                    scatter(k - 2, b).wait()

                transpose_scale(b)
                scatter(k, b).start()

                @pl.when(tt < half - 1)
                def _():
                    gather(k + 2, b).start()

            return carry

        lax.fori_loop(0, half, ring, 0)

        for b in range(2):
            scatter(n_chunks - 2 + b, b).wait()

    return emb_kernel


def kernel(tokens, table):
    B0, T = tokens.shape
    V, D = table.shape
    flat = tokens.T.reshape(B0 * T).astype(jnp.int32)
    out4 = _build(B0, T, V, D)(table, flat)
    out = (
        out4.reshape(T, D // 8, B0 // 128, 8, 128)
        .transpose(2, 4, 0, 1, 3)
        .reshape(B0, T, D)
    )
    return out
